# Initial kernel scaffold; baseline (speedup 1.0000x reference)
import jax
import jax.numpy as jnp
from jax.experimental import pallas as pl


def kernel(x, W0, W1, W2, W3):
    raise NotImplementedError("write your pallas kernel here")


# speedup vs baseline: 5.2774x; 5.2774x over previous
import functools

import jax
import jax.numpy as jnp
from jax import lax
from jax.experimental import pallas as pl
from jax.experimental.pallas import tpu as pltpu
from jax.experimental.pallas import tpu_sc as plsc

NC = 2
NS = 16
NW = NC * NS
LANES = 16
CH = 128
NBUF = 2


def kernel(x, W0, W1, W2, W3):
    B, S = x.shape
    V, D = W0.shape
    N = B * S
    per_w = N // NW
    n_ch = per_w // CH

    xf = x.reshape(-1).astype(jnp.int32)

    mesh = plsc.VectorSubcoreMesh(core_axis_name="c", subcore_axis_name="s")

    @functools.partial(
        pl.kernel,
        mesh=mesh,
        out_type=jax.ShapeDtypeStruct((N, D), jnp.float32),
        compiler_params=pltpu.CompilerParams(use_tc_tiling_on_sc=False),
        scratch_types=[
            pltpu.VMEM((per_w,), jnp.int32),
            *([pltpu.VMEM((CH, D), jnp.float32)] * (NBUF * 4)),
            *([pltpu.VMEM((CH, D), jnp.float32)] * NBUF),
            *([pltpu.SemaphoreType.DMA] * (NBUF * 2)),
        ],
    )
    def sc_avg(x_hbm, w0_hbm, w1_hbm, w2_hbm, w3_hbm, out_hbm,
               idx_v,
               b00, b01, b02, b03, b10, b11, b12, b13,
               ob0, ob1, gsem0, gsem1, ssem0, ssem1):
        wid = lax.axis_index("s") * NC + lax.axis_index("c")
        pltpu.sync_copy(x_hbm.at[pl.ds(wid * per_w, per_w)], idx_v)

        tabs = (w0_hbm, w1_hbm, w2_hbm, w3_hbm)
        bufs = ((b00, b01, b02, b03), (b10, b11, b12, b13))
        obufs = (ob0, ob1)
        gsems = (gsem0, gsem1)
        ssems = (ssem0, ssem1)

        def gathers(c, s):
            idx = idx_v.at[pl.ds(c * CH, CH)]
            return [pltpu.make_async_copy(tabs[t].at[idx], bufs[s][t],
                                          gsems[s]) for t in range(4)]

        for cp in gathers(0, 0):
            cp.start()
        for cp in gathers(1, 1):
            cp.start()

        def pair_body(p, carry):
            for s in range(NBUF):
                c = p * NBUF + s
                bs, ob = bufs[s], obufs[s]
                for cp in gathers(c, s):
                    cp.wait()

                @pl.when(c >= NBUF)
                def _():
                    pltpu.make_async_copy(
                        ob, out_hbm.at[pl.ds(0, CH)], ssems[s]).wait()

                def row_body(i, carry2):
                    for j in range(D // LANES):
                        sl = pl.ds(j * LANES, LANES)
                        ob[i, sl] = ((bs[0][i, sl] + bs[1][i, sl])
                                     + (bs[2][i, sl] + bs[3][i, sl])) * 0.25
                    return carry2

                lax.fori_loop(0, CH, row_body, 0, unroll=4)

                pltpu.make_async_copy(
                    ob, out_hbm.at[pl.ds((wid * n_ch + c) * CH, CH)],
                    ssems[s]).start()

                @pl.when(c + NBUF < n_ch)
                def _():
                    for cp in gathers(c + NBUF, s):
                        cp.start()
            return carry

        lax.fori_loop(0, n_ch // NBUF, pair_body, 0)
        for s in range(NBUF):
            pltpu.make_async_copy(
                obufs[s], out_hbm.at[pl.ds(0, CH)], ssems[s]).wait()

    out = sc_avg(xf, W0, W1, W2, W3)
    return out.reshape(B, S, D)


# speedup vs baseline: 6.3484x; 1.2029x over previous
import functools

import jax
import jax.numpy as jnp
from jax import lax
from jax.experimental import pallas as pl
from jax.experimental.pallas import tpu as pltpu
from jax.experimental.pallas import tpu_sc as plsc

NC = 2
NS = 16
NW = NC * NS
LANES = 16
CH = 128
NBUF = 2


def kernel(x, W0, W1, W2, W3):
    B, S = x.shape
    V, D = W0.shape
    N = B * S
    per_w = N // NW
    n_ch = per_w // CH

    xf = x.reshape(-1).astype(jnp.int32)

    mesh = plsc.VectorSubcoreMesh(core_axis_name="c", subcore_axis_name="s")

    @functools.partial(
        pl.kernel,
        mesh=mesh,
        out_type=jax.ShapeDtypeStruct((N, D), jnp.float32),
        compiler_params=pltpu.CompilerParams(use_tc_tiling_on_sc=False),
        scratch_types=[
            pltpu.VMEM((per_w,), jnp.int32),
            *([pltpu.VMEM((CH, D), jnp.float32)] * NBUF),
            *([pltpu.VMEM((CH, D), jnp.float32)] * NBUF),
            *([pltpu.SemaphoreType.DMA] * (NBUF * 2)),
        ],
    )
    def sc_avg(x_hbm, w0_hbm, w1_hbm, w2_hbm, w3_hbm, out_hbm,
               idx_v, ac0, ac1, ob0, ob1, gsem0, gsem1, ssem0, ssem1):
        wid = lax.axis_index("s") * NC + lax.axis_index("c")
        pltpu.sync_copy(x_hbm.at[pl.ds(wid * per_w, per_w)], idx_v)

        tabs = (w0_hbm, w1_hbm, w2_hbm, w3_hbm)
        accs = (ac0, ac1)
        obufs = (ob0, ob1)
        gsems = (gsem0, gsem1)
        ssems = (ssem0, ssem1)
        zeros = jnp.zeros((LANES,), jnp.float32)

        def zero_acc(ac):
            def zbody(i, carry):
                for j in range(D // LANES):
                    ac[i, pl.ds(j * LANES, LANES)] = zeros
                return carry
            lax.fori_loop(0, CH, zbody, 0, unroll=8)

        def fire(c, s):
            idx = idx_v.at[pl.ds(c * CH, CH)]
            for t in range(4):
                pltpu.async_copy(tabs[t].at[idx], accs[s], gsems[s],
                                 add=True)

        for s in range(NBUF):
            zero_acc(accs[s])
            fire(s, s)

        def pair_body(p, carry):
            for s in range(NBUF):
                c = p * NBUF + s
                ac, ob = accs[s], obufs[s]
                idx0 = idx_v.at[pl.ds(0, CH)]
                for _ in range(4):
                    pltpu.make_async_copy(tabs[0].at[idx0], ac,
                                          gsems[s]).wait()

                @pl.when(c >= NBUF)
                def _():
                    pltpu.make_async_copy(
                        ob, out_hbm.at[pl.ds(0, CH)], ssems[s]).wait()

                def row_body(i, carry2):
                    for j in range(D // LANES):
                        sl = pl.ds(j * LANES, LANES)
                        v = ac[i, sl]
                        ac[i, sl] = zeros
                        ob[i, sl] = v * 0.25
                    return carry2

                lax.fori_loop(0, CH, row_body, 0, unroll=4)

                pltpu.make_async_copy(
                    ob, out_hbm.at[pl.ds((wid * n_ch + c) * CH, CH)],
                    ssems[s]).start()

                @pl.when(c + NBUF < n_ch)
                def _():
                    fire(c + NBUF, s)
            return carry

        lax.fori_loop(0, n_ch // NBUF, pair_body, 0)
        for s in range(NBUF):
            pltpu.make_async_copy(
                obufs[s], out_hbm.at[pl.ds(0, CH)], ssems[s]).wait()

    out = sc_avg(xf, W0, W1, W2, W3)
    return out.reshape(B, S, D)
